# baseline (speedup 1.0000x reference)
import dataclasses
import functools

import jax
import jax.numpy as jnp
from jax import lax
from jax.experimental import pallas as pl
from jax.experimental.pallas import tpu as pltpu
from jax.experimental.pallas import tpu_sc as plsc

N = 10000
E = 320000
D = 128
NCLS = 40
NC = 2
NS = 16
NW = NC * NS
R = 320
NPAD = NW * R
CH = 16000
NCHUNK = E // CH
QD = 2048
CAPW = E + QD
GB = 128

_MESH = plsc.VectorSubcoreMesh(core_axis_name="c", subcore_axis_name="s")

_SC_PARAMS = pltpu.CompilerParams()
if "needs_layout_passes" in pltpu.CompilerParams.__dataclass_fields__:
  _SC_PARAMS = dataclasses.replace(_SC_PARAMS, needs_layout_passes=False)


def _sc_build(rows, cols):
  @functools.partial(
      pl.kernel,
      out_type=(
          jax.ShapeDtypeStruct((NW * CAPW,), jnp.int32),
          jax.ShapeDtypeStruct((NW * 16,), jnp.int32),
          jax.ShapeDtypeStruct((NW * R,), jnp.float32),
      ),
      mesh=_MESH,
      scratch_types=[
          pltpu.VMEM((2, CH), jnp.int32),
          pltpu.VMEM((2, CH), jnp.int32),
          pltpu.VMEM((QD + 16,), jnp.int32),
          pltpu.VMEM((R,), jnp.float32),
          pltpu.SemaphoreType.DMA,
          pltpu.SemaphoreType.DMA,
      ],
      compiler_params=_SC_PARAMS,
  )
  def k(r_hbm, c_hbm, lists_hbm, lens_hbm, deg_hbm, rows_v, cols_v, q_v, deg_v,
        semr, semc):
    wid = lax.axis_index("c") * NS + lax.axis_index("s")
    lo = wid * R

    @pl.loop(0, R // 16)
    def _(j):
      deg_v[pl.ds(j * 16, 16)] = jnp.zeros((16,), jnp.float32)

    lovec = jnp.full((16,), lo, jnp.int32)
    hivec = lovec + R
    ones = jnp.ones((16,), jnp.float32)

    def start_fetch(ch, b):
      pltpu.async_copy(r_hbm.at[pl.ds(ch * CH, CH)], rows_v.at[b], semr)
      pltpu.async_copy(c_hbm.at[pl.ds(ch * CH, CH)], cols_v.at[b], semc)

    def wait_fetch(ch, b):
      pltpu.make_async_copy(r_hbm.at[pl.ds(ch * CH, CH)], rows_v.at[b], semr).wait()
      pltpu.make_async_copy(c_hbm.at[pl.ds(ch * CH, CH)], cols_v.at[b], semc).wait()

    start_fetch(0, 0)

    def chunk_pair(p, carry):
      for b in (0, 1):
        ch = 2 * p + b

        @pl.when(ch + 1 < NCHUNK)
        def _():
          start_fetch(ch + 1, 1 - b)

        wait_fetch(ch, b)

        def body(j, qo):
          qpos, outpos = qo
          c = cols_v[b, pl.ds(j * 16, 16)]
          r = rows_v[b, pl.ds(j * 16, 16)]
          m = (c >= lovec) & (c < hivec)
          cl = jnp.where(m, c - lovec, 0)
          plsc.addupdate_scatter(deg_v, [cl], ones, mask=m)
          packed = (r << 9) | cl
          plsc.store_compressed(q_v.at[pl.ds(qpos, 16)], packed, mask=m)
          qpos = qpos + plsc.all_reduce_population_count(m)[0]

          def drain(args):
            qp, op = args
            pltpu.sync_copy(q_v.at[pl.ds(0, QD)],
                            lists_hbm.at[pl.ds(pl.multiple_of(wid * CAPW + op, 8), QD)])
            q_v[pl.ds(0, 16)] = q_v[pl.ds(QD, 16)]
            return (qp - QD, op + QD)

          return lax.cond(qpos >= QD, drain, lambda a: a, (qpos, outpos))

        carry = lax.fori_loop(0, CH // 16, body, carry, unroll=4)
      return carry

    qpos, outpos = lax.fori_loop(0, NCHUNK // 2, chunk_pair,
                                 (jnp.int32(0), jnp.int32(0)), unroll=False)
    pltpu.sync_copy(q_v.at[pl.ds(0, QD)],
                    lists_hbm.at[pl.ds(pl.multiple_of(wid * CAPW + outpos, 8), QD)])
    q_v[pl.ds(0, 16)] = jnp.full((16,), outpos + qpos, jnp.int32)
    pltpu.sync_copy(q_v.at[pl.ds(0, 16)], lens_hbm.at[pl.ds(pl.multiple_of(wid * 16, 8), 16)])
    pltpu.sync_copy(deg_v, deg_hbm.at[pl.ds(pl.multiple_of(wid * R, 8), R)])

  return k(rows, cols)


def _sc_aggregate(y, lists, lens):
  NG = GB // 16
  ACCROWS = R + 1

  @functools.partial(
      pl.kernel,
      out_type=jax.ShapeDtypeStruct((NPAD, D), jnp.float32),
      mesh=_MESH,
      scratch_types=[
          pltpu.VMEM((ACCROWS, D), jnp.float32),
          pltpu.VMEM((2, GB), jnp.int32),
          pltpu.VMEM((2, GB), jnp.int32),
          pltpu.VMEM((2, GB), jnp.int32),
          pltpu.VMEM((2, GB, D), jnp.float32),
          pltpu.VMEM((16,), jnp.int32),
          pltpu.SemaphoreType.DMA,
          pltpu.SemaphoreType.DMA,
          pltpu.SemaphoreType.DMA,
          pltpu.SemaphoreType.DMA,
      ],
      compiler_params=_SC_PARAMS,
  )
  def k(y_hbm, lists_hbm, lens_hbm, seg_hbm, acc_v, lq_v, idx_v, clq_v, g_v,
        len_v, seml0, seml1, semg0, semg1):
    semls = (seml0, seml1)
    semgs = (semg0, semg1)
    wid = lax.axis_index("c") * NS + lax.axis_index("s")
    pltpu.sync_copy(y_hbm.at[pl.ds(wid * R, R), :], acc_v.at[pl.ds(0, R), :])
    pltpu.sync_copy(lens_hbm.at[pl.ds(pl.multiple_of(wid * 16, 8), 16)], len_v)
    ln = len_v[pl.ds(0, 16)][0]
    lnvec = jnp.full((16,), ln, jnp.int32)
    ntrip = (ln + (GB - 1)) // GB

    def list_src(t):
      return lists_hbm.at[pl.ds(pl.multiple_of(wid * CAPW + t * GB, 8), GB)]

    def start_list(t, b):
      pltpu.async_copy(list_src(t), lq_v.at[b], semls[b])

    def unpack_and_gather(t, b):
      remv = lnvec - t * GB
      for j in range(NG):
        pk = lq_v[b, pl.ds(j * 16, 16)]
        lanes = lax.iota(jnp.int32, 16) + (j * 16)
        valid = lanes < remv
        idx_v[b, pl.ds(j * 16, 16)] = jnp.where(valid, pk >> 9, lanes)
        clq_v[b, pl.ds(j * 16, 16)] = jnp.where(valid, pk & 511, R)
      pltpu.async_copy(y_hbm.at[idx_v.at[b]], g_v.at[b], semgs[b])

    def process(b):
      @pl.loop(0, NG)
      def _(g):
        clvec = clq_v[b, pl.ds(g * 16, 16)]
        for e in range(16):
          ee = g * 16 + e
          cl = clvec[e]
          for d in range(D // 16):
            s = pl.ds(d * 16, 16)
            acc_v[cl, s] = jnp.maximum(acc_v[cl, s], g_v[b, ee, s])

    @pl.when(ntrip > 0)
    def _():
      pltpu.sync_copy(list_src(0), lq_v.at[0])
      unpack_and_gather(0, 0)

      @pl.when(ntrip > 1)
      def _():
        start_list(1, 1)

      def pair(p, _):
        for par in (0, 1):
          t = 2 * p + par

          @pl.when(t < ntrip)
          def _():
            @pl.when(t + 2 < ntrip)
            def _():
              start_list(t + 2, par)

            @pl.when(t + 1 < ntrip)
            def _():
              pltpu.make_async_copy(
                  list_src(t + 1), lq_v.at[1 - par], semls[1 - par]).wait()
              unpack_and_gather(t + 1, 1 - par)

            pltpu.make_async_copy(
                y_hbm.at[idx_v.at[par]], g_v.at[par], semgs[par]).wait()
            process(par)
        return 0

      lax.fori_loop(0, (ntrip + 1) // 2, pair, 0, unroll=False)

    pltpu.sync_copy(acc_v.at[pl.ds(0, R), :], seg_hbm.at[pl.ds(wid * R, R), :])

  return k(y, lists, lens)


_BLK = 1024


def _tc1(x_pad, W1, deg):
  def body(x_ref, w_ref, deg_ref, y_ref):
    dinv = lax.rsqrt(deg_ref[...] + 1.0)
    y_ref[...] = dinv * jnp.dot(
        x_ref[...], w_ref[...], preferred_element_type=jnp.float32)

  return pl.pallas_call(
      body,
      grid=(NPAD // _BLK,),
      in_specs=[
          pl.BlockSpec((_BLK, D), lambda i: (i, 0)),
          pl.BlockSpec((D, D), lambda i: (0, 0)),
          pl.BlockSpec((_BLK, 1), lambda i: (i, 0)),
      ],
      out_specs=pl.BlockSpec((_BLK, D), lambda i: (i, 0)),
      out_shape=jax.ShapeDtypeStruct((NPAD, D), jnp.float32),
  )(x_pad, W1, deg)


def _tc2(seg, deg, b, W):
  def body(seg_ref, deg_ref, b_ref, w_ref, y_ref):
    dinv = lax.rsqrt(deg_ref[...] + 1.0)
    h = jnp.maximum(dinv * seg_ref[...] + b_ref[...], 0.0)
    y_ref[...] = dinv * jnp.dot(
        h, w_ref[...], preferred_element_type=jnp.float32)

  return pl.pallas_call(
      body,
      grid=(NPAD // _BLK,),
      in_specs=[
          pl.BlockSpec((_BLK, D), lambda i: (i, 0)),
          pl.BlockSpec((_BLK, 1), lambda i: (i, 0)),
          pl.BlockSpec((1, D), lambda i: (0, 0)),
          pl.BlockSpec((D, D), lambda i: (0, 0)),
      ],
      out_specs=pl.BlockSpec((_BLK, D), lambda i: (i, 0)),
      out_shape=jax.ShapeDtypeStruct((NPAD, D), jnp.float32),
  )(seg, deg, b, W)


def _tc3(seg, deg, b2, W3p, b3p):
  def body(seg_ref, deg_ref, b2_ref, w_ref, b3_ref, o_ref):
    dinv = lax.rsqrt(deg_ref[...] + 1.0)
    h = jnp.maximum(dinv * seg_ref[...] + b2_ref[...], 0.0)
    logits = jnp.dot(h, w_ref[...], preferred_element_type=jnp.float32) + b3_ref[...]
    colm = lax.broadcasted_iota(jnp.int32, (_BLK, D), 1) < NCLS
    neg = jnp.float32(-1e30)
    lm = jnp.max(jnp.where(colm, logits, neg), axis=1, keepdims=True)
    se = jnp.sum(jnp.where(colm, jnp.exp(logits - lm), 0.0), axis=1, keepdims=True)
    o_ref[...] = logits - lm - jnp.log(se)

  return pl.pallas_call(
      body,
      grid=(NPAD // _BLK,),
      in_specs=[
          pl.BlockSpec((_BLK, D), lambda i: (i, 0)),
          pl.BlockSpec((_BLK, 1), lambda i: (i, 0)),
          pl.BlockSpec((1, D), lambda i: (0, 0)),
          pl.BlockSpec((D, D), lambda i: (0, 0)),
          pl.BlockSpec((1, D), lambda i: (0, 0)),
      ],
      out_specs=pl.BlockSpec((_BLK, D), lambda i: (i, 0)),
      out_shape=jax.ShapeDtypeStruct((NPAD, D), jnp.float32),
  )(seg, deg, b2, W3p, b3p)


def kernel(x, edge_index, W1, b1, W2, b2, W3, b3):
  x_pad = jnp.pad(x, ((0, NPAD - N), (0, 0)))
  lists, lens, deg_blocks = _sc_build(edge_index[0], edge_index[1])
  deg = deg_blocks.reshape(NPAD, 1)
  y1 = _tc1(x_pad, W1, deg)
  seg1 = _sc_aggregate(y1, lists, lens)
  y2 = _tc2(seg1, deg, b1.reshape(1, D), W2)
  seg2 = _sc_aggregate(y2, lists, lens)
  W3p = jnp.pad(W3, ((0, 0), (0, D - NCLS)))
  b3p = jnp.pad(b3, (0, D - NCLS)).reshape(1, D)
  out = _tc3(seg2, deg, b2.reshape(1, D), W3p, b3p)
  return out[:N, :NCLS]


# speedup vs baseline: 1.2901x; 1.2901x over previous
import dataclasses
import functools

import jax
import jax.numpy as jnp
from jax import lax
from jax.experimental import pallas as pl
from jax.experimental.pallas import tpu as pltpu
from jax.experimental.pallas import tpu_sc as plsc

N = 10000
E = 320000
D = 128
NCLS = 40
NC = 2
NS = 16
NW = NC * NS
R = 320
NPAD = NW * R
CH = 16000
NCHUNK = E // CH
QD = 2048
CAPW = E + QD
GB = 128
_GRP = 4
_SEG = 10

_MESH = plsc.VectorSubcoreMesh(core_axis_name="c", subcore_axis_name="s")

_SC_PARAMS = pltpu.CompilerParams()
if "needs_layout_passes" in pltpu.CompilerParams.__dataclass_fields__:
  _SC_PARAMS = dataclasses.replace(_SC_PARAMS, needs_layout_passes=False)


def _sc_build(rows, cols):
  @functools.partial(
      pl.kernel,
      out_type=(
          jax.ShapeDtypeStruct((NW * CAPW,), jnp.int32),
          jax.ShapeDtypeStruct((NW * 16,), jnp.int32),
          jax.ShapeDtypeStruct((NW * R,), jnp.float32),
      ),
      mesh=_MESH,
      scratch_types=[
          pltpu.VMEM((2, CH), jnp.int32),
          pltpu.VMEM((2, CH), jnp.int32),
          pltpu.VMEM((QD + 16 * _SEG * _GRP + 16,), jnp.int32),
          pltpu.VMEM((R,), jnp.float32),
          pltpu.SemaphoreType.DMA,
          pltpu.SemaphoreType.DMA,
      ],
      compiler_params=_SC_PARAMS,
  )
  def k(r_hbm, c_hbm, lists_hbm, lens_hbm, deg_hbm, rows_v, cols_v, q_v, deg_v,
        semr, semc):
    wid = lax.axis_index("c") * NS + lax.axis_index("s")
    lo = wid * R

    @pl.loop(0, R // 16)
    def _(j):
      deg_v[pl.ds(j * 16, 16)] = jnp.zeros((16,), jnp.float32)

    lovec = jnp.full((16,), lo, jnp.int32)
    hivec = lovec + R
    ones = jnp.ones((16,), jnp.float32)

    def start_fetch(ch, b):
      pltpu.async_copy(r_hbm.at[pl.ds(ch * CH, CH)], rows_v.at[b], semr)
      pltpu.async_copy(c_hbm.at[pl.ds(ch * CH, CH)], cols_v.at[b], semc)

    def wait_fetch(ch, b):
      pltpu.make_async_copy(r_hbm.at[pl.ds(ch * CH, CH)], rows_v.at[b], semr).wait()
      pltpu.make_async_copy(c_hbm.at[pl.ds(ch * CH, CH)], cols_v.at[b], semc).wait()

    start_fetch(0, 0)

    def chunk_pair(p, carry):
      for b in (0, 1):
        ch = 2 * p + b

        @pl.when(ch + 1 < NCHUNK)
        def _():
          start_fetch(ch + 1, 1 - b)

        wait_fetch(ch, b)

        def segment(si, qo):
          qpos, outpos = qo

          def group(gi, qp):
            base = (si * _SEG + gi) * (_GRP * 16)
            ms, pks = [], []
            for u in range(_GRP):
              off = base + u * 16
              c = cols_v[b, pl.ds(off, 16)]
              r = rows_v[b, pl.ds(off, 16)]
              m = (c >= lovec) & (c < hivec)
              cl = jnp.where(m, c - lovec, 0)
              plsc.addupdate_scatter(deg_v, [cl], ones, mask=m)
              ms.append(m)
              pks.append((r << 9) | cl)
            ps = [plsc.all_reduce_population_count(m)[0] for m in ms]
            for u in range(_GRP):
              plsc.store_compressed(q_v.at[pl.ds(qp, 16)], pks[u], mask=ms[u])
              qp = qp + ps[u]
            return qp

          qpos = lax.fori_loop(0, _SEG, group, qpos, unroll=True)

          def drain(args):
            qp, op = args
            pltpu.sync_copy(q_v.at[pl.ds(0, QD)],
                            lists_hbm.at[pl.ds(pl.multiple_of(wid * CAPW + op, 8), QD)])
            nmove = (qp - QD + 15) // 16

            def mv(i, _):
              q_v[pl.ds(i * 16, 16)] = q_v[pl.ds(QD + i * 16, 16)]
              return 0

            lax.fori_loop(0, nmove, mv, 0)
            return (qp - QD, op + QD)

          return lax.cond(qpos >= QD, drain, lambda a: a, (qpos, outpos))

        carry = lax.fori_loop(0, (CH // 16) // (_SEG * _GRP), segment, carry,
                              unroll=False)
      return carry

    qpos, outpos = lax.fori_loop(0, NCHUNK // 2, chunk_pair,
                                 (jnp.int32(0), jnp.int32(0)), unroll=False)
    pltpu.sync_copy(q_v.at[pl.ds(0, QD)],
                    lists_hbm.at[pl.ds(pl.multiple_of(wid * CAPW + outpos, 8), QD)])
    q_v[pl.ds(0, 16)] = jnp.full((16,), outpos + qpos, jnp.int32)
    pltpu.sync_copy(q_v.at[pl.ds(0, 16)], lens_hbm.at[pl.ds(pl.multiple_of(wid * 16, 8), 16)])
    pltpu.sync_copy(deg_v, deg_hbm.at[pl.ds(pl.multiple_of(wid * R, 8), R)])

  return k(rows, cols)


def _sc_aggregate(y, lists, lens):
  NG = GB // 16
  ACCROWS = R + 1

  @functools.partial(
      pl.kernel,
      out_type=jax.ShapeDtypeStruct((NPAD, D), jnp.float32),
      mesh=_MESH,
      scratch_types=[
          pltpu.VMEM((ACCROWS, D), jnp.float32),
          pltpu.VMEM((2, GB), jnp.int32),
          pltpu.VMEM((2, GB), jnp.int32),
          pltpu.VMEM((2, GB), jnp.int32),
          pltpu.VMEM((2, GB, D), jnp.float32),
          pltpu.VMEM((16,), jnp.int32),
          pltpu.SemaphoreType.DMA,
          pltpu.SemaphoreType.DMA,
          pltpu.SemaphoreType.DMA,
          pltpu.SemaphoreType.DMA,
      ],
      compiler_params=_SC_PARAMS,
  )
  def k(y_hbm, lists_hbm, lens_hbm, seg_hbm, acc_v, lq_v, idx_v, clq_v, g_v,
        len_v, seml0, seml1, semg0, semg1):
    semls = (seml0, seml1)
    semgs = (semg0, semg1)
    wid = lax.axis_index("c") * NS + lax.axis_index("s")
    pltpu.sync_copy(y_hbm.at[pl.ds(wid * R, R), :], acc_v.at[pl.ds(0, R), :])
    pltpu.sync_copy(lens_hbm.at[pl.ds(pl.multiple_of(wid * 16, 8), 16)], len_v)
    ln = len_v[pl.ds(0, 16)][0]
    lnvec = jnp.full((16,), ln, jnp.int32)
    ntrip = (ln + (GB - 1)) // GB

    def list_src(t):
      return lists_hbm.at[pl.ds(pl.multiple_of(wid * CAPW + t * GB, 8), GB)]

    def start_list(t, b):
      pltpu.async_copy(list_src(t), lq_v.at[b], semls[b])

    def unpack_and_gather(t, b):
      remv = lnvec - t * GB
      for j in range(NG):
        pk = lq_v[b, pl.ds(j * 16, 16)]
        lanes = lax.iota(jnp.int32, 16) + (j * 16)
        valid = lanes < remv
        idx_v[b, pl.ds(j * 16, 16)] = jnp.where(valid, pk >> 9, lanes)
        clq_v[b, pl.ds(j * 16, 16)] = jnp.where(valid, pk & 511, R)
      pltpu.async_copy(y_hbm.at[idx_v.at[b]], g_v.at[b], semgs[b])

    def process(b):
      @pl.loop(0, NG)
      def _(g):
        clvec = clq_v[b, pl.ds(g * 16, 16)]
        for e in range(16):
          ee = g * 16 + e
          cl = clvec[e]
          for d in range(D // 16):
            s = pl.ds(d * 16, 16)
            acc_v[cl, s] = jnp.maximum(acc_v[cl, s], g_v[b, ee, s])

    @pl.when(ntrip > 0)
    def _():
      pltpu.sync_copy(list_src(0), lq_v.at[0])
      unpack_and_gather(0, 0)

      @pl.when(ntrip > 1)
      def _():
        start_list(1, 1)

      def pair(p, _):
        for par in (0, 1):
          t = 2 * p + par

          @pl.when(t < ntrip)
          def _():
            @pl.when(t + 2 < ntrip)
            def _():
              start_list(t + 2, par)

            @pl.when(t + 1 < ntrip)
            def _():
              pltpu.make_async_copy(
                  list_src(t + 1), lq_v.at[1 - par], semls[1 - par]).wait()
              unpack_and_gather(t + 1, 1 - par)

            pltpu.make_async_copy(
                y_hbm.at[idx_v.at[par]], g_v.at[par], semgs[par]).wait()
            process(par)
        return 0

      lax.fori_loop(0, (ntrip + 1) // 2, pair, 0, unroll=False)

    pltpu.sync_copy(acc_v.at[pl.ds(0, R), :], seg_hbm.at[pl.ds(wid * R, R), :])

  return k(y, lists, lens)


_BLK = 1024


def _tc1(x_pad, W1, deg):
  def body(x_ref, w_ref, deg_ref, y_ref):
    dinv = lax.rsqrt(deg_ref[...] + 1.0)
    y_ref[...] = dinv * jnp.dot(
        x_ref[...], w_ref[...], preferred_element_type=jnp.float32)

  return pl.pallas_call(
      body,
      grid=(NPAD // _BLK,),
      in_specs=[
          pl.BlockSpec((_BLK, D), lambda i: (i, 0)),
          pl.BlockSpec((D, D), lambda i: (0, 0)),
          pl.BlockSpec((_BLK, 1), lambda i: (i, 0)),
      ],
      out_specs=pl.BlockSpec((_BLK, D), lambda i: (i, 0)),
      out_shape=jax.ShapeDtypeStruct((NPAD, D), jnp.float32),
  )(x_pad, W1, deg)


def _tc2(seg, deg, b, W):
  def body(seg_ref, deg_ref, b_ref, w_ref, y_ref):
    dinv = lax.rsqrt(deg_ref[...] + 1.0)
    h = jnp.maximum(dinv * seg_ref[...] + b_ref[...], 0.0)
    y_ref[...] = dinv * jnp.dot(
        h, w_ref[...], preferred_element_type=jnp.float32)

  return pl.pallas_call(
      body,
      grid=(NPAD // _BLK,),
      in_specs=[
          pl.BlockSpec((_BLK, D), lambda i: (i, 0)),
          pl.BlockSpec((_BLK, 1), lambda i: (i, 0)),
          pl.BlockSpec((1, D), lambda i: (0, 0)),
          pl.BlockSpec((D, D), lambda i: (0, 0)),
      ],
      out_specs=pl.BlockSpec((_BLK, D), lambda i: (i, 0)),
      out_shape=jax.ShapeDtypeStruct((NPAD, D), jnp.float32),
  )(seg, deg, b, W)


def _tc3(seg, deg, b2, W3p, b3p):
  def body(seg_ref, deg_ref, b2_ref, w_ref, b3_ref, o_ref):
    dinv = lax.rsqrt(deg_ref[...] + 1.0)
    h = jnp.maximum(dinv * seg_ref[...] + b2_ref[...], 0.0)
    logits = jnp.dot(h, w_ref[...], preferred_element_type=jnp.float32) + b3_ref[...]
    colm = lax.broadcasted_iota(jnp.int32, (_BLK, D), 1) < NCLS
    neg = jnp.float32(-1e30)
    lm = jnp.max(jnp.where(colm, logits, neg), axis=1, keepdims=True)
    se = jnp.sum(jnp.where(colm, jnp.exp(logits - lm), 0.0), axis=1, keepdims=True)
    o_ref[...] = logits - lm - jnp.log(se)

  return pl.pallas_call(
      body,
      grid=(NPAD // _BLK,),
      in_specs=[
          pl.BlockSpec((_BLK, D), lambda i: (i, 0)),
          pl.BlockSpec((_BLK, 1), lambda i: (i, 0)),
          pl.BlockSpec((1, D), lambda i: (0, 0)),
          pl.BlockSpec((D, D), lambda i: (0, 0)),
          pl.BlockSpec((1, D), lambda i: (0, 0)),
      ],
      out_specs=pl.BlockSpec((_BLK, D), lambda i: (i, 0)),
      out_shape=jax.ShapeDtypeStruct((NPAD, D), jnp.float32),
  )(seg, deg, b2, W3p, b3p)


def kernel(x, edge_index, W1, b1, W2, b2, W3, b3):
  x_pad = jnp.pad(x, ((0, NPAD - N), (0, 0)))
  lists, lens, deg_blocks = _sc_build(edge_index[0], edge_index[1])
  deg = deg_blocks.reshape(NPAD, 1)
  y1 = _tc1(x_pad, W1, deg)
  seg1 = _sc_aggregate(y1, lists, lens)
  y2 = _tc2(seg1, deg, b1.reshape(1, D), W2)
  seg2 = _sc_aggregate(y2, lists, lens)
  W3p = jnp.pad(W3, ((0, 0), (0, D - NCLS)))
  b3p = jnp.pad(b3, (0, D - NCLS)).reshape(1, D)
  out = _tc3(seg2, deg, b2.reshape(1, D), W3p, b3p)
  return out[:N, :NCLS]
